# baseline (speedup 1.0000x reference)
import functools
import math

import jax
import jax.numpy as jnp
from jax import lax
from jax.experimental import pallas as pl
from jax.experimental.pallas import tpu as pltpu
from jax.experimental.pallas import tpu_sc as plsc

D_MODEL = 768
LANES = 16
SCALE = math.sqrt(float(D_MODEL))

_B = 4 * 2048
_NW = 32
_BPW = _B // _NW
_CHUNK = 32
_NCHUNK = _BPW // _CHUNK
_NBUF = 4


def _emb_body(x_hbm, table_hbm, out_hbm, idx_v, rows_v, *sems):
    in_sems = sems[:_NBUF]
    out_sems = sems[_NBUF:]
    wid = lax.axis_index("s") * 2 + lax.axis_index("c")
    base = wid * _BPW
    scale = jnp.full((LANES,), SCALE, dtype=jnp.float32)

    pltpu.sync_copy(x_hbm.at[wid], idx_v)

    def start_gather(g):
        b = g % _NBUF
        return pltpu.async_copy(
            table_hbm.at[idx_v.at[g]], rows_v.at[b], in_sems[b])

    gathers = [None] * _NCHUNK
    stores = [None] * _NCHUNK
    store_waited = [False] * _NCHUNK

    for g in range(_NBUF - 1):
        gathers[g] = start_gather(g)

    for g in range(_NCHUNK):
        b = g % _NBUF
        nx = g + _NBUF - 1
        if nx < _NCHUNK:
            if g >= 1:
                stores[g - 1].wait()
                store_waited[g - 1] = True
            gathers[nx] = start_gather(nx)

        gathers[g].wait()
        buf = rows_v.at[b]

        @plsc.parallel_loop(0, _CHUNK)
        def _(r):
            for j in range(D_MODEL // LANES):
                sl = pl.ds(j * LANES, LANES)
                buf[r, sl] = buf[r, sl] * scale

        stores[g] = pltpu.async_copy(
            buf, out_hbm.at[pl.ds(base + g * _CHUNK, _CHUNK)], out_sems[b])

    for g in range(_NCHUNK):
        if not store_waited[g]:
            stores[g].wait()


def kernel(x, table):
    x_split = x.reshape(_NW, _NCHUNK, _CHUNK).astype(jnp.int32)
    mesh = plsc.VectorSubcoreMesh(core_axis_name="c", subcore_axis_name="s")
    run = functools.partial(
        pl.kernel,
        mesh=mesh,
        out_type=jax.ShapeDtypeStruct((_B, D_MODEL), jnp.float32),
        scratch_types=[
            pltpu.VMEM((_NCHUNK, _CHUNK), jnp.int32),
            pltpu.VMEM((_NBUF, _CHUNK, D_MODEL), jnp.float32),
        ] + [pltpu.SemaphoreType.DMA] * (2 * _NBUF),
    )(_emb_body)
    out = run(x_split, table)
    return out.reshape(x.shape[0], x.shape[1], D_MODEL)
